# baseline (speedup 1.0000x reference)
import functools

import jax
import jax.numpy as jnp
from jax import lax
from jax.experimental import pallas as pl
from jax.experimental.pallas import tpu as pltpu
from jax.experimental.pallas import tpu_sc as plsc

N_FACTORS = 32
BATCH = 16384
N_ROWS = 1000000
NC = 2
NS = 16
NW = NC * NS
B_PER_W = BATCH // NW
PER_TEC = B_PER_W * N_FACTORS

DT_C = 4096
DT_G = DT_C // 128
DT_GRID = (N_ROWS + DT_C - 1) // DT_C
ROWS_LIN = DT_GRID * (DT_C // 4)


def _detile_body(x_ref, o_ref):
    for q in range(4):
        xq = x_ref[:, 1024 * q: 1024 * (q + 1)]
        o_ref[:, 32 * q: 32 * (q + 1)] = xq.T


def _detile(xT):
    return pl.pallas_call(
        _detile_body,
        grid=(DT_GRID,),
        in_specs=[pl.BlockSpec((N_FACTORS, DT_C), lambda i: (0, i))],
        out_specs=pl.BlockSpec((DT_C // 4, 128), lambda i: (i, 0)),
        out_shape=jax.ShapeDtypeStruct((ROWS_LIN, 128), jnp.float32),
    )(xT)


SC_BOUND = 999424
SC_WINDOWS = SC_BOUND // 512
SC_W_PER_TEC = SC_WINDOWS // NW
ROWS_MAIN = (SC_BOUND // 4096) * 1024


def _detile_sc_body(mT, mlin, slab_v, st_v, sem_in, sem_out):
    wid = lax.axis_index("s") * NC + lax.axis_index("c")
    iota = lax.iota(jnp.int32, 16)

    def window_coords(t):
        wnd = wid * SC_W_PER_TEC + t
        k = wnd >> 3
        v0 = (wnd & 7) * 128
        return k, v0

    def fire_in(t, slot):
        k, v0 = window_coords(t)
        for q in range(4):
            c0 = pl.multiple_of(k * 4096 + q * 1024 + v0, 128)
            for tr in range(4):
                pltpu.async_copy(
                    mT.at[pl.ds(8 * tr, 8), pl.ds(c0, 128)],
                    slab_v.at[slot, q, pl.ds(8 * tr, 8), :], sem_in)

    def wait_in(t, slot):
        k, v0 = window_coords(t)
        for q in range(4):
            c0 = pl.multiple_of(k * 4096 + q * 1024 + v0, 128)
            for tr in range(4):
                pltpu.make_async_copy(
                    mT.at[pl.ds(8 * tr, 8), pl.ds(c0, 128)],
                    slab_v.at[slot, q, pl.ds(8 * tr, 8), :], sem_in).wait()

    def out_dst(t):
        k, v0 = window_coords(t)
        return mlin.at[pl.ds(pl.multiple_of(k * 1024 + v0, 128), 128), :]

    fire_in(0, 0)

    def body(t, carry):
        slot = t & 1

        @pl.when(t + 1 < SC_W_PER_TEC)
        def _():
            fire_in(t + 1, slot ^ 1)

        wait_in(t, slot)

        @pl.when(t >= 2)
        def _():
            pltpu.make_async_copy(st_v.at[slot], out_dst(t), sem_out).wait()

        for q in range(4):
            for f in range(N_FACTORS):
                col = jnp.full((16,), 32 * q + f, jnp.int32)
                for g in range(8):
                    vals = slab_v[slot, q, f, pl.ds(g * 16, 16)]
                    plsc.store_scatter(st_v.at[slot],
                                       [g * 16 + iota, col], vals)
        pltpu.async_copy(st_v.at[slot], out_dst(t), sem_out)
        return carry

    lax.fori_loop(0, SC_W_PER_TEC, body, 0)
    for t in (SC_W_PER_TEC - 2, SC_W_PER_TEC - 1):
        pltpu.make_async_copy(st_v.at[t & 1], out_dst(t), sem_out).wait()


_detile_sc_cache = []


def _detile_sc(*args):
    if not _detile_sc_cache:
        _detile_sc_cache.append(functools.partial(
            pl.kernel,
            mesh=plsc.VectorSubcoreMesh(core_axis_name="c",
                                        subcore_axis_name="s"),
            out_type=[jax.ShapeDtypeStruct((ROWS_MAIN, 128), jnp.float32)],
            scratch_types=[
                pltpu.VMEM((2, 4, N_FACTORS, 128), jnp.float32),
                pltpu.VMEM((2, 128, 128), jnp.float32),
                pltpu.SemaphoreType.DMA,
                pltpu.SemaphoreType.DMA,
            ],
            compiler_params=pltpu.CompilerParams(needs_layout_passes=False),
        )(_detile_sc_body))
    return _detile_sc_cache[0](*args)[0]


def _detile_tail(mT):
    return pl.pallas_call(
        _detile_body,
        grid=(1,),
        in_specs=[pl.BlockSpec((N_FACTORS, DT_C),
                               lambda i: (0, SC_BOUND // DT_C))],
        out_specs=pl.BlockSpec((DT_C // 4, 128), lambda i: (0, 0)),
        out_shape=jax.ShapeDtypeStruct((DT_C // 4, 128), jnp.float32),
    )(mT)


def _gather_body(ulin, mlin, mtail, users_hbm, movies_hbm, uout, mout,
                 uidx_v, midx_v, ulane_v, mlane_v, mflag_v, rowidx_v, raw_v,
                 uex_v, mex_v, sem):
    wid = lax.axis_index("s") * NC + lax.axis_index("c")
    base = wid * B_PER_W
    pltpu.sync_copy(users_hbm.at[pl.ds(base, B_PER_W)], uidx_v)
    pltpu.sync_copy(movies_hbm.at[pl.ds(base, B_PER_W)], midx_v)

    iota = lax.iota(jnp.int32, 16)

    for g in range(B_PER_W // 16):
        sl = pl.ds(g * 16, 16)
        uvec = uidx_v[sl]
        rowidx_v[0, g // 8, pl.ds((g % 8) * 16, 16)] = \
            ((uvec >> 12) << 10) + (uvec & 1023)
        ulane_v[sl] = ((uvec >> 10) & 3) * 32
        mvec = midx_v[sl]
        tail = mvec >= SC_BOUND
        mrow = ((mvec >> 12) << 10) + (mvec & 1023)
        rowidx_v[1, g // 8, pl.ds((g % 8) * 16, 16)] = \
            jnp.where(tail, 0, mrow)
        rowidx_v[2, g // 8, pl.ds((g % 8) * 16, 16)] = \
            jnp.where(tail, mvec - SC_BOUND, 0)
        mlane_v[sl] = ((mvec >> 10) & 3) * 32
        mflag_v[sl] = jnp.where(tail, 1, 0)

    def gather_pass(table_sel, lin, lanes_v, ex_v, flag_val):
        copies = []
        for j in range(B_PER_W // 128):
            copies.append(pltpu.async_copy(
                lin.at[rowidx_v.at[table_sel, j]],
                raw_v.at[pl.ds(j * 128, 128)], sem))
        for c in copies:
            c.wait()

        def extract_f(f, carry):
            for g in range(B_PER_W // 16):
                rows16 = g * 16 + iota
                cols16 = lanes_v[pl.ds(g * 16, 16)] + f
                vals = plsc.load_gather(raw_v, [rows16, cols16])
                pos = rows16 * N_FACTORS + f
                if flag_val is None:
                    plsc.store_scatter(ex_v, [pos], vals)
                else:
                    mask = mflag_v[pl.ds(g * 16, 16)] == flag_val
                    plsc.store_scatter(ex_v, [pos], vals, mask=mask)
            return carry

        lax.fori_loop(0, N_FACTORS, extract_f, 0)

    gather_pass(0, ulin, ulane_v, uex_v, None)
    gather_pass(1, mlin, mlane_v, mex_v, 0)
    gather_pass(2, mtail, mlane_v, mex_v, 1)
    pltpu.sync_copy(uex_v, uout.at[wid])
    pltpu.sync_copy(mex_v, mout.at[wid])


_gather_cache = []


def _gather(*args):
    if not _gather_cache:
        _gather_cache.append(functools.partial(
            pl.kernel,
            mesh=plsc.VectorSubcoreMesh(core_axis_name="c",
                                        subcore_axis_name="s"),
            out_type=[
                jax.ShapeDtypeStruct((NW, PER_TEC), jnp.float32),
                jax.ShapeDtypeStruct((NW, PER_TEC), jnp.float32),
            ],
            scratch_types=[
                pltpu.VMEM((B_PER_W,), jnp.int32),
                pltpu.VMEM((B_PER_W,), jnp.int32),
                pltpu.VMEM((B_PER_W,), jnp.int32),
                pltpu.VMEM((B_PER_W,), jnp.int32),
                pltpu.VMEM((B_PER_W,), jnp.int32),
                pltpu.VMEM((3, B_PER_W // 128, 128), jnp.int32),
                pltpu.VMEM((B_PER_W, 128), jnp.float32),
                pltpu.VMEM((PER_TEC,), jnp.float32),
                pltpu.VMEM((PER_TEC,), jnp.float32),
                pltpu.SemaphoreType.DMA,
            ],
            compiler_params=pltpu.CompilerParams(needs_layout_passes=False),
        )(_gather_body))
    return _gather_cache[0](*args)


def _mlp_body(u_ref, m_ref, w1u_ref, w1m_ref, b1_ref, w2_ref, b2_ref,
              wf_ref, bf_ref, o_ref):
    x = jnp.dot(u_ref[...], w1u_ref[...], preferred_element_type=jnp.float32)
    x = x + jnp.dot(m_ref[...], w1m_ref[...], preferred_element_type=jnp.float32)
    h = jnp.maximum(x + b1_ref[...], 0.0)
    h = jnp.maximum(
        jnp.dot(h, w2_ref[...], preferred_element_type=jnp.float32)
        + b2_ref[...], 0.0)
    s = jnp.dot(h, wf_ref[...], preferred_element_type=jnp.float32) + bf_ref[...]
    o_ref[...] = jax.nn.sigmoid(s) * 4.5 + 0.5


def _mlp(u, m, w1u, w1m, b1, w2, b2, wf, bf, block_b=2048):
    nb = BATCH // block_b
    wspec = lambda shape: pl.BlockSpec(shape, lambda i: (0, 0))
    return pl.pallas_call(
        _mlp_body,
        grid=(nb,),
        in_specs=[
            pl.BlockSpec((block_b, N_FACTORS), lambda i: (i, 0)),
            pl.BlockSpec((block_b, N_FACTORS), lambda i: (i, 0)),
            wspec(w1u.shape),
            wspec(w1m.shape),
            wspec(b1.shape),
            wspec(w2.shape),
            wspec(b2.shape),
            wspec(wf.shape),
            wspec(bf.shape),
        ],
        out_specs=pl.BlockSpec((block_b, 1), lambda i: (i, 0)),
        out_shape=jax.ShapeDtypeStruct((BATCH, 1), jnp.float32),
    )(u, m, w1u, w1m, b1, w2, b2, wf, bf)


@jax.jit
def kernel(users, movies, user_emb, movie_emb, W1, b1, W2, b2, Wf, bf):
    u_lin = _detile(user_emb.T)
    m_lin = _detile_sc(movie_emb.T)
    m_tail = _detile_tail(movie_emb.T)
    u_raw, m_raw = _gather(u_lin, m_lin, m_tail,
                           users.astype(jnp.int32), movies.astype(jnp.int32))
    u_rows = u_raw.reshape(BATCH, N_FACTORS)
    m_rows = m_raw.reshape(BATCH, N_FACTORS)
    w1u = W1[:N_FACTORS]
    w1m = W1[N_FACTORS:]
    return _mlp(u_rows, m_rows, w1u, w1m,
                b1.reshape(1, -1), W2, b2.reshape(1, -1),
                Wf, bf.reshape(1, 1))


# speedup vs baseline: 2.4723x; 2.4723x over previous
import functools

import jax
import jax.numpy as jnp
from jax import lax
from jax.experimental import pallas as pl
from jax.experimental.pallas import tpu as pltpu
from jax.experimental.pallas import tpu_sc as plsc

N_FACTORS = 32
BATCH = 16384
N_ROWS = 1000000
NC = 2
NS = 16
NW = NC * NS
B_PER_W = BATCH // NW
PER_TEC = B_PER_W * N_FACTORS

DT_C = 4096
DT_G = DT_C // 128
DT_GRID = (N_ROWS + DT_C - 1) // DT_C
ROWS_LIN = DT_GRID * (DT_C // 4)


def _detile_body(x_ref, o_ref):
    for q in range(4):
        xq = x_ref[:, 1024 * q: 1024 * (q + 1)]
        o_ref[:, 32 * q: 32 * (q + 1)] = xq.T


def _detile(xT):
    return pl.pallas_call(
        _detile_body,
        grid=(DT_GRID,),
        in_specs=[pl.BlockSpec((N_FACTORS, DT_C), lambda i: (0, i))],
        out_specs=pl.BlockSpec((DT_C // 4, 128), lambda i: (i, 0)),
        out_shape=jax.ShapeDtypeStruct((ROWS_LIN, 128), jnp.float32),
    )(xT)


SC_BOUND = 638976
SC_WINDOWS = SC_BOUND // 512
SC_W_PER_TEC = SC_WINDOWS // NW
ROWS_MAIN = (SC_BOUND // 4096) * 1024
TAIL_BLOCK0 = SC_BOUND // DT_C
TAIL_BLOCKS = (N_ROWS - SC_BOUND + DT_C - 1) // DT_C
ROWS_TAIL = TAIL_BLOCKS * (DT_C // 4)


def _detile_sc_body(mT, mlin, slab_v, st_v, sem_in, sem_out):
    wid = lax.axis_index("s") * NC + lax.axis_index("c")
    iota = lax.iota(jnp.int32, 16)

    def window_coords(t):
        wnd = wid * SC_W_PER_TEC + t
        k = wnd >> 3
        v0 = (wnd & 7) * 128
        return k, v0

    def fire_in(t, slot):
        k, v0 = window_coords(t)
        for q in range(4):
            c0 = pl.multiple_of(k * 4096 + q * 1024 + v0, 128)
            for tr in range(4):
                pltpu.async_copy(
                    mT.at[pl.ds(8 * tr, 8), pl.ds(c0, 128)],
                    slab_v.at[slot, q, pl.ds(8 * tr, 8), :], sem_in)

    def wait_in(t, slot):
        k, v0 = window_coords(t)
        for q in range(4):
            c0 = pl.multiple_of(k * 4096 + q * 1024 + v0, 128)
            for tr in range(4):
                pltpu.make_async_copy(
                    mT.at[pl.ds(8 * tr, 8), pl.ds(c0, 128)],
                    slab_v.at[slot, q, pl.ds(8 * tr, 8), :], sem_in).wait()

    def out_dst(t):
        k, v0 = window_coords(t)
        return mlin.at[pl.ds(pl.multiple_of(k * 1024 + v0, 128), 128), :]

    fire_in(0, 0)

    def body(t, carry):
        slot = t & 1

        @pl.when(t + 1 < SC_W_PER_TEC)
        def _():
            fire_in(t + 1, slot ^ 1)

        wait_in(t, slot)

        @pl.when(t >= 2)
        def _():
            pltpu.make_async_copy(st_v.at[slot], out_dst(t), sem_out).wait()

        for q in range(4):
            for f in range(N_FACTORS):
                col = jnp.full((16,), 32 * q + f, jnp.int32)
                for g in range(8):
                    vals = slab_v[slot, q, f, pl.ds(g * 16, 16)]
                    plsc.store_scatter(st_v.at[slot],
                                       [g * 16 + iota, col], vals)
        pltpu.async_copy(st_v.at[slot], out_dst(t), sem_out)
        return carry

    lax.fori_loop(0, SC_W_PER_TEC, body, 0)
    for t in (SC_W_PER_TEC - 2, SC_W_PER_TEC - 1):
        pltpu.make_async_copy(st_v.at[t & 1], out_dst(t), sem_out).wait()


_detile_sc_cache = []


def _detile_sc(*args):
    if not _detile_sc_cache:
        _detile_sc_cache.append(functools.partial(
            pl.kernel,
            mesh=plsc.VectorSubcoreMesh(core_axis_name="c",
                                        subcore_axis_name="s"),
            out_type=[jax.ShapeDtypeStruct((ROWS_MAIN, 128), jnp.float32)],
            scratch_types=[
                pltpu.VMEM((2, 4, N_FACTORS, 128), jnp.float32),
                pltpu.VMEM((2, 128, 128), jnp.float32),
                pltpu.SemaphoreType.DMA,
                pltpu.SemaphoreType.DMA,
            ],
            compiler_params=pltpu.CompilerParams(needs_layout_passes=False),
        )(_detile_sc_body))
    return _detile_sc_cache[0](*args)[0]


def _detile_tail(mT):
    return pl.pallas_call(
        _detile_body,
        grid=(TAIL_BLOCKS,),
        in_specs=[pl.BlockSpec((N_FACTORS, DT_C),
                               lambda i: (0, i + TAIL_BLOCK0))],
        out_specs=pl.BlockSpec((DT_C // 4, 128), lambda i: (i, 0)),
        out_shape=jax.ShapeDtypeStruct((ROWS_TAIL, 128), jnp.float32),
    )(mT)


def _gather_body(ulin, mlin, mtail, users_hbm, movies_hbm, uout, mout,
                 uidx_v, midx_v, ulane_v, mlane_v, mflag_v, rowidx_v, raw_v,
                 uex_v, mex_v, sem):
    wid = lax.axis_index("s") * NC + lax.axis_index("c")
    base = wid * B_PER_W
    pltpu.sync_copy(users_hbm.at[pl.ds(base, B_PER_W)], uidx_v)
    pltpu.sync_copy(movies_hbm.at[pl.ds(base, B_PER_W)], midx_v)

    iota = lax.iota(jnp.int32, 16)

    for g in range(B_PER_W // 16):
        sl = pl.ds(g * 16, 16)
        uvec = uidx_v[sl]
        rowidx_v[0, g // 8, pl.ds((g % 8) * 16, 16)] = \
            ((uvec >> 12) << 10) + (uvec & 1023)
        ulane_v[sl] = ((uvec >> 10) & 3) * 32
        mvec = midx_v[sl]
        tail = mvec >= SC_BOUND
        mrow = ((mvec >> 12) << 10) + (mvec & 1023)
        trow = (((mvec >> 12) - TAIL_BLOCK0) << 10) + (mvec & 1023)
        spread = g * 16 + iota
        rowidx_v[1, g // 8, pl.ds((g % 8) * 16, 16)] = \
            jnp.where(tail, spread, mrow)
        rowidx_v[2, g // 8, pl.ds((g % 8) * 16, 16)] = \
            jnp.where(tail, trow, spread)
        mlane_v[sl] = ((mvec >> 10) & 3) * 32
        mflag_v[sl] = jnp.where(tail, 1, 0)

    def gather_pass(table_sel, lin, lanes_v, ex_v, flag_val):
        copies = []
        for j in range(B_PER_W // 128):
            copies.append(pltpu.async_copy(
                lin.at[rowidx_v.at[table_sel, j]],
                raw_v.at[pl.ds(j * 128, 128)], sem))
        for c in copies:
            c.wait()

        def extract_f(f, carry):
            for g in range(B_PER_W // 16):
                rows16 = g * 16 + iota
                cols16 = lanes_v[pl.ds(g * 16, 16)] + f
                vals = plsc.load_gather(raw_v, [rows16, cols16])
                pos = rows16 * N_FACTORS + f
                if flag_val is None:
                    plsc.store_scatter(ex_v, [pos], vals)
                else:
                    mask = mflag_v[pl.ds(g * 16, 16)] == flag_val
                    plsc.store_scatter(ex_v, [pos], vals, mask=mask)
            return carry

        lax.fori_loop(0, N_FACTORS, extract_f, 0)

    gather_pass(0, ulin, ulane_v, uex_v, None)
    gather_pass(1, mlin, mlane_v, mex_v, 0)
    gather_pass(2, mtail, mlane_v, mex_v, 1)
    pltpu.sync_copy(uex_v, uout.at[wid])
    pltpu.sync_copy(mex_v, mout.at[wid])


_gather_cache = []


def _gather(*args):
    if not _gather_cache:
        _gather_cache.append(functools.partial(
            pl.kernel,
            mesh=plsc.VectorSubcoreMesh(core_axis_name="c",
                                        subcore_axis_name="s"),
            out_type=[
                jax.ShapeDtypeStruct((NW, PER_TEC), jnp.float32),
                jax.ShapeDtypeStruct((NW, PER_TEC), jnp.float32),
            ],
            scratch_types=[
                pltpu.VMEM((B_PER_W,), jnp.int32),
                pltpu.VMEM((B_PER_W,), jnp.int32),
                pltpu.VMEM((B_PER_W,), jnp.int32),
                pltpu.VMEM((B_PER_W,), jnp.int32),
                pltpu.VMEM((B_PER_W,), jnp.int32),
                pltpu.VMEM((3, B_PER_W // 128, 128), jnp.int32),
                pltpu.VMEM((B_PER_W, 128), jnp.float32),
                pltpu.VMEM((PER_TEC,), jnp.float32),
                pltpu.VMEM((PER_TEC,), jnp.float32),
                pltpu.SemaphoreType.DMA,
            ],
            compiler_params=pltpu.CompilerParams(needs_layout_passes=False),
        )(_gather_body))
    return _gather_cache[0](*args)


def _mlp_body(u_ref, m_ref, w1u_ref, w1m_ref, b1_ref, w2_ref, b2_ref,
              wf_ref, bf_ref, o_ref):
    x = jnp.dot(u_ref[...], w1u_ref[...], preferred_element_type=jnp.float32)
    x = x + jnp.dot(m_ref[...], w1m_ref[...], preferred_element_type=jnp.float32)
    h = jnp.maximum(x + b1_ref[...], 0.0)
    h = jnp.maximum(
        jnp.dot(h, w2_ref[...], preferred_element_type=jnp.float32)
        + b2_ref[...], 0.0)
    s = jnp.dot(h, wf_ref[...], preferred_element_type=jnp.float32) + bf_ref[...]
    o_ref[...] = jax.nn.sigmoid(s) * 4.5 + 0.5


def _mlp(u, m, w1u, w1m, b1, w2, b2, wf, bf, block_b=2048):
    nb = BATCH // block_b
    wspec = lambda shape: pl.BlockSpec(shape, lambda i: (0, 0))
    return pl.pallas_call(
        _mlp_body,
        grid=(nb,),
        in_specs=[
            pl.BlockSpec((block_b, N_FACTORS), lambda i: (i, 0)),
            pl.BlockSpec((block_b, N_FACTORS), lambda i: (i, 0)),
            wspec(w1u.shape),
            wspec(w1m.shape),
            wspec(b1.shape),
            wspec(w2.shape),
            wspec(b2.shape),
            wspec(wf.shape),
            wspec(bf.shape),
        ],
        out_specs=pl.BlockSpec((block_b, 1), lambda i: (i, 0)),
        out_shape=jax.ShapeDtypeStruct((BATCH, 1), jnp.float32),
    )(u, m, w1u, w1m, b1, w2, b2, wf, bf)


@jax.jit
def kernel(users, movies, user_emb, movie_emb, W1, b1, W2, b2, Wf, bf):
    u_lin = _detile(user_emb.T)
    m_lin = _detile_sc(movie_emb.T)
    m_tail = _detile_tail(movie_emb.T)
    u_raw, m_raw = _gather(u_lin, m_lin, m_tail,
                           users.astype(jnp.int32), movies.astype(jnp.int32))
    u_rows = u_raw.reshape(BATCH, N_FACTORS)
    m_rows = m_raw.reshape(BATCH, N_FACTORS)
    w1u = W1[:N_FACTORS]
    w1m = W1[N_FACTORS:]
    return _mlp(u_rows, m_rows, w1u, w1m,
                b1.reshape(1, -1), W2, b2.reshape(1, -1),
                Wf, bf.reshape(1, 1))


# speedup vs baseline: 2.5193x; 1.0190x over previous
import functools

import jax
import jax.numpy as jnp
from jax import lax
from jax.experimental import pallas as pl
from jax.experimental.pallas import tpu as pltpu
from jax.experimental.pallas import tpu_sc as plsc

N_FACTORS = 32
BATCH = 16384
N_ROWS = 1000000
NC = 2
NS = 16
NW = NC * NS
B_PER_W = BATCH // NW
PER_TEC = B_PER_W * N_FACTORS

DT_C = 4096
DT_G = DT_C // 128
DT_GRID = (N_ROWS + DT_C - 1) // DT_C
ROWS_LIN = DT_GRID * (DT_C // 4)


def _detile_body(x_ref, o_ref):
    for q in range(4):
        xq = x_ref[:, 1024 * q: 1024 * (q + 1)]
        o_ref[:, 32 * q: 32 * (q + 1)] = xq.T


def _detile(xT):
    return pl.pallas_call(
        _detile_body,
        grid=(DT_GRID,),
        in_specs=[pl.BlockSpec((N_FACTORS, DT_C), lambda i: (0, i))],
        out_specs=pl.BlockSpec((DT_C // 4, 128), lambda i: (i, 0)),
        out_shape=jax.ShapeDtypeStruct((ROWS_LIN, 128), jnp.float32),
    )(xT)


SC_BOUND = 638976
SC_WINDOWS = SC_BOUND // 512
SC_W_PER_TEC = SC_WINDOWS // NW
ROWS_MAIN = (SC_BOUND // 4096) * 1024
TAIL_BLOCK0 = SC_BOUND // DT_C
TAIL_BLOCKS = (N_ROWS - SC_BOUND + DT_C - 1) // DT_C
ROWS_TAIL = TAIL_BLOCKS * (DT_C // 4)


def _detile_sc_body(mT, mlin, slab_v, st_v, sem_in, sem_out):
    wid = lax.axis_index("s") * NC + lax.axis_index("c")
    iota = lax.iota(jnp.int32, 16)

    def window_coords(t):
        wnd = wid * SC_W_PER_TEC + t
        k = wnd >> 3
        v0 = (wnd & 7) * 128
        return k, v0

    def fire_in(t, slot):
        k, v0 = window_coords(t)
        for q in range(4):
            c0 = pl.multiple_of(k * 4096 + q * 1024 + v0, 128)
            for tr in range(4):
                pltpu.async_copy(
                    mT.at[pl.ds(8 * tr, 8), pl.ds(c0, 128)],
                    slab_v.at[slot, q, pl.ds(8 * tr, 8), :], sem_in)

    def wait_in(t, slot):
        k, v0 = window_coords(t)
        for q in range(4):
            c0 = pl.multiple_of(k * 4096 + q * 1024 + v0, 128)
            for tr in range(4):
                pltpu.make_async_copy(
                    mT.at[pl.ds(8 * tr, 8), pl.ds(c0, 128)],
                    slab_v.at[slot, q, pl.ds(8 * tr, 8), :], sem_in).wait()

    def out_dst(t):
        k, v0 = window_coords(t)
        return mlin.at[pl.ds(pl.multiple_of(k * 1024 + v0, 128), 128), :]

    fire_in(0, 0)

    def body(t, carry):
        slot = t & 1

        @pl.when(t + 1 < SC_W_PER_TEC)
        def _():
            fire_in(t + 1, slot ^ 1)

        wait_in(t, slot)

        @pl.when(t >= 2)
        def _():
            pltpu.make_async_copy(st_v.at[slot], out_dst(t), sem_out).wait()

        for q in range(4):
            for f in range(N_FACTORS):
                col = jnp.full((16,), 32 * q + f, jnp.int32)
                vals = [slab_v[slot, q, f, pl.ds(g * 16, 16)]
                        for g in range(8)]
                for g in range(8):
                    plsc.store_scatter(st_v.at[slot],
                                       [g * 16 + iota, col], vals[g])
        pltpu.async_copy(st_v.at[slot], out_dst(t), sem_out)
        return carry

    lax.fori_loop(0, SC_W_PER_TEC, body, 0)
    for t in (SC_W_PER_TEC - 2, SC_W_PER_TEC - 1):
        pltpu.make_async_copy(st_v.at[t & 1], out_dst(t), sem_out).wait()


_detile_sc_cache = []


def _detile_sc(*args):
    if not _detile_sc_cache:
        _detile_sc_cache.append(functools.partial(
            pl.kernel,
            mesh=plsc.VectorSubcoreMesh(core_axis_name="c",
                                        subcore_axis_name="s"),
            out_type=[jax.ShapeDtypeStruct((ROWS_MAIN, 128), jnp.float32)],
            scratch_types=[
                pltpu.VMEM((2, 4, N_FACTORS, 128), jnp.float32),
                pltpu.VMEM((2, 128, 128), jnp.float32),
                pltpu.SemaphoreType.DMA,
                pltpu.SemaphoreType.DMA,
            ],
            compiler_params=pltpu.CompilerParams(needs_layout_passes=False),
        )(_detile_sc_body))
    return _detile_sc_cache[0](*args)[0]


def _detile_tail(mT):
    return pl.pallas_call(
        _detile_body,
        grid=(TAIL_BLOCKS,),
        in_specs=[pl.BlockSpec((N_FACTORS, DT_C),
                               lambda i: (0, i + TAIL_BLOCK0))],
        out_specs=pl.BlockSpec((DT_C // 4, 128), lambda i: (i, 0)),
        out_shape=jax.ShapeDtypeStruct((ROWS_TAIL, 128), jnp.float32),
    )(mT)


def _gather_body(ulin, mlin, mtail, users_hbm, movies_hbm, uout, mout,
                 uidx_v, midx_v, ulane_v, mlane_v, mflag_v, rowidx_v, raw_v,
                 uex_v, mex_v, sem):
    wid = lax.axis_index("s") * NC + lax.axis_index("c")
    base = wid * B_PER_W
    pltpu.sync_copy(users_hbm.at[pl.ds(base, B_PER_W)], uidx_v)
    pltpu.sync_copy(movies_hbm.at[pl.ds(base, B_PER_W)], midx_v)

    iota = lax.iota(jnp.int32, 16)

    for g in range(B_PER_W // 16):
        sl = pl.ds(g * 16, 16)
        uvec = uidx_v[sl]
        rowidx_v[0, g // 8, pl.ds((g % 8) * 16, 16)] = \
            ((uvec >> 12) << 10) + (uvec & 1023)
        ulane_v[sl] = ((uvec >> 10) & 3) * 32
        mvec = midx_v[sl]
        tail = mvec >= SC_BOUND
        mrow = ((mvec >> 12) << 10) + (mvec & 1023)
        trow = (((mvec >> 12) - TAIL_BLOCK0) << 10) + (mvec & 1023)
        spread = g * 16 + iota
        rowidx_v[1, g // 8, pl.ds((g % 8) * 16, 16)] = \
            jnp.where(tail, spread, mrow)
        rowidx_v[2, g // 8, pl.ds((g % 8) * 16, 16)] = \
            jnp.where(tail, trow, spread)
        mlane_v[sl] = ((mvec >> 10) & 3) * 32
        mflag_v[sl] = jnp.where(tail, 1, 0)

    def gather_pass(table_sel, lin, lanes_v, ex_v, flag_val):
        copies = []
        for j in range(B_PER_W // 128):
            copies.append(pltpu.async_copy(
                lin.at[rowidx_v.at[table_sel, j]],
                raw_v.at[pl.ds(j * 128, 128)], sem))
        for c in copies:
            c.wait()

        def extract_f(f, carry):
            ng = B_PER_W // 16
            gathered = []
            for g in range(ng):
                rows16 = g * 16 + iota
                cols16 = lanes_v[pl.ds(g * 16, 16)] + f
                gathered.append(plsc.load_gather(raw_v, [rows16, cols16]))
            for g in range(ng):
                pos = (g * 16 + iota) * N_FACTORS + f
                if flag_val is None:
                    plsc.store_scatter(ex_v, [pos], gathered[g])
                else:
                    mask = mflag_v[pl.ds(g * 16, 16)] == flag_val
                    plsc.store_scatter(ex_v, [pos], gathered[g], mask=mask)
            return carry

        lax.fori_loop(0, N_FACTORS, extract_f, 0)

    gather_pass(0, ulin, ulane_v, uex_v, None)
    gather_pass(1, mlin, mlane_v, mex_v, 0)
    gather_pass(2, mtail, mlane_v, mex_v, 1)
    pltpu.sync_copy(uex_v, uout.at[wid])
    pltpu.sync_copy(mex_v, mout.at[wid])


_gather_cache = []


def _gather(*args):
    if not _gather_cache:
        _gather_cache.append(functools.partial(
            pl.kernel,
            mesh=plsc.VectorSubcoreMesh(core_axis_name="c",
                                        subcore_axis_name="s"),
            out_type=[
                jax.ShapeDtypeStruct((NW, PER_TEC), jnp.float32),
                jax.ShapeDtypeStruct((NW, PER_TEC), jnp.float32),
            ],
            scratch_types=[
                pltpu.VMEM((B_PER_W,), jnp.int32),
                pltpu.VMEM((B_PER_W,), jnp.int32),
                pltpu.VMEM((B_PER_W,), jnp.int32),
                pltpu.VMEM((B_PER_W,), jnp.int32),
                pltpu.VMEM((B_PER_W,), jnp.int32),
                pltpu.VMEM((3, B_PER_W // 128, 128), jnp.int32),
                pltpu.VMEM((B_PER_W, 128), jnp.float32),
                pltpu.VMEM((PER_TEC,), jnp.float32),
                pltpu.VMEM((PER_TEC,), jnp.float32),
                pltpu.SemaphoreType.DMA,
            ],
            compiler_params=pltpu.CompilerParams(needs_layout_passes=False),
        )(_gather_body))
    return _gather_cache[0](*args)


def _mlp_body(u_ref, m_ref, w1u_ref, w1m_ref, b1_ref, w2_ref, b2_ref,
              wf_ref, bf_ref, o_ref):
    x = jnp.dot(u_ref[...], w1u_ref[...], preferred_element_type=jnp.float32)
    x = x + jnp.dot(m_ref[...], w1m_ref[...], preferred_element_type=jnp.float32)
    h = jnp.maximum(x + b1_ref[...], 0.0)
    h = jnp.maximum(
        jnp.dot(h, w2_ref[...], preferred_element_type=jnp.float32)
        + b2_ref[...], 0.0)
    s = jnp.dot(h, wf_ref[...], preferred_element_type=jnp.float32) + bf_ref[...]
    o_ref[...] = jax.nn.sigmoid(s) * 4.5 + 0.5


def _mlp(u, m, w1u, w1m, b1, w2, b2, wf, bf, block_b=2048):
    nb = BATCH // block_b
    wspec = lambda shape: pl.BlockSpec(shape, lambda i: (0, 0))
    return pl.pallas_call(
        _mlp_body,
        grid=(nb,),
        in_specs=[
            pl.BlockSpec((block_b, N_FACTORS), lambda i: (i, 0)),
            pl.BlockSpec((block_b, N_FACTORS), lambda i: (i, 0)),
            wspec(w1u.shape),
            wspec(w1m.shape),
            wspec(b1.shape),
            wspec(w2.shape),
            wspec(b2.shape),
            wspec(wf.shape),
            wspec(bf.shape),
        ],
        out_specs=pl.BlockSpec((block_b, 1), lambda i: (i, 0)),
        out_shape=jax.ShapeDtypeStruct((BATCH, 1), jnp.float32),
    )(u, m, w1u, w1m, b1, w2, b2, wf, bf)


@jax.jit
def kernel(users, movies, user_emb, movie_emb, W1, b1, W2, b2, Wf, bf):
    u_lin = _detile(user_emb.T)
    m_lin = _detile_sc(movie_emb.T)
    m_tail = _detile_tail(movie_emb.T)
    u_raw, m_raw = _gather(u_lin, m_lin, m_tail,
                           users.astype(jnp.int32), movies.astype(jnp.int32))
    u_rows = u_raw.reshape(BATCH, N_FACTORS)
    m_rows = m_raw.reshape(BATCH, N_FACTORS)
    w1u = W1[:N_FACTORS]
    w1m = W1[N_FACTORS:]
    return _mlp(u_rows, m_rows, w1u, w1m,
                b1.reshape(1, -1), W2, b2.reshape(1, -1),
                Wf, bf.reshape(1, 1))


# speedup vs baseline: 2.5394x; 1.0080x over previous
import functools

import jax
import jax.numpy as jnp
from jax import lax
from jax.experimental import pallas as pl
from jax.experimental.pallas import tpu as pltpu
from jax.experimental.pallas import tpu_sc as plsc

N_FACTORS = 32
BATCH = 16384
N_ROWS = 1000000
NC = 2
NS = 16
NW = NC * NS
B_PER_W = BATCH // NW
PER_TEC = B_PER_W * N_FACTORS

DT_C = 4096
DT_G = DT_C // 128
DT_GRID = (N_ROWS + DT_C - 1) // DT_C
ROWS_LIN = DT_GRID * (DT_C // 4)


def _detile_body(x_ref, o_ref):
    for q in range(4):
        xq = x_ref[:, 1024 * q: 1024 * (q + 1)]
        o_ref[:, 32 * q: 32 * (q + 1)] = xq.T


def _detile(xT):
    return pl.pallas_call(
        _detile_body,
        grid=(DT_GRID,),
        in_specs=[pl.BlockSpec((N_FACTORS, DT_C), lambda i: (0, i))],
        out_specs=pl.BlockSpec((DT_C // 4, 128), lambda i: (i, 0)),
        out_shape=jax.ShapeDtypeStruct((ROWS_LIN, 128), jnp.float32),
    )(xT)


SC_BOUND = 671744
SC_WINDOWS = SC_BOUND // 512
SC_W_PER_TEC = SC_WINDOWS // NW
ROWS_MAIN = (SC_BOUND // 4096) * 1024
TAIL_BLOCK0 = SC_BOUND // DT_C
TAIL_BLOCKS = (N_ROWS - SC_BOUND + DT_C - 1) // DT_C
ROWS_TAIL = TAIL_BLOCKS * (DT_C // 4)


def _detile_sc_body(mT, mlin, slab_v, st_v, sem_in, sem_out):
    wid = lax.axis_index("s") * NC + lax.axis_index("c")
    iota = lax.iota(jnp.int32, 16)

    def window_coords(t):
        wnd = wid * SC_W_PER_TEC + t
        k = wnd >> 3
        v0 = (wnd & 7) * 128
        return k, v0

    def fire_in(t, slot):
        k, v0 = window_coords(t)
        for q in range(4):
            c0 = pl.multiple_of(k * 4096 + q * 1024 + v0, 128)
            for tr in range(4):
                pltpu.async_copy(
                    mT.at[pl.ds(8 * tr, 8), pl.ds(c0, 128)],
                    slab_v.at[slot, q, pl.ds(8 * tr, 8), :], sem_in)

    def wait_in(t, slot):
        k, v0 = window_coords(t)
        for q in range(4):
            c0 = pl.multiple_of(k * 4096 + q * 1024 + v0, 128)
            for tr in range(4):
                pltpu.make_async_copy(
                    mT.at[pl.ds(8 * tr, 8), pl.ds(c0, 128)],
                    slab_v.at[slot, q, pl.ds(8 * tr, 8), :], sem_in).wait()

    def out_dst(t):
        k, v0 = window_coords(t)
        return mlin.at[pl.ds(pl.multiple_of(k * 1024 + v0, 128), 128), :]

    fire_in(0, 0)

    def body(t, carry):
        slot = t & 1

        @pl.when(t + 1 < SC_W_PER_TEC)
        def _():
            fire_in(t + 1, slot ^ 1)

        wait_in(t, slot)

        @pl.when(t >= 2)
        def _():
            pltpu.make_async_copy(st_v.at[slot], out_dst(t), sem_out).wait()

        for q in range(4):
            for f in range(N_FACTORS):
                col = jnp.full((16,), 32 * q + f, jnp.int32)
                vals = [slab_v[slot, q, f, pl.ds(g * 16, 16)]
                        for g in range(8)]
                for g in range(8):
                    plsc.store_scatter(st_v.at[slot],
                                       [g * 16 + iota, col], vals[g])
        pltpu.async_copy(st_v.at[slot], out_dst(t), sem_out)
        return carry

    lax.fori_loop(0, SC_W_PER_TEC, body, 0)
    for t in (SC_W_PER_TEC - 2, SC_W_PER_TEC - 1):
        pltpu.make_async_copy(st_v.at[t & 1], out_dst(t), sem_out).wait()


_detile_sc_cache = []


def _detile_sc(*args):
    if not _detile_sc_cache:
        _detile_sc_cache.append(functools.partial(
            pl.kernel,
            mesh=plsc.VectorSubcoreMesh(core_axis_name="c",
                                        subcore_axis_name="s"),
            out_type=[jax.ShapeDtypeStruct((ROWS_MAIN, 128), jnp.float32)],
            scratch_types=[
                pltpu.VMEM((2, 4, N_FACTORS, 128), jnp.float32),
                pltpu.VMEM((2, 128, 128), jnp.float32),
                pltpu.SemaphoreType.DMA,
                pltpu.SemaphoreType.DMA,
            ],
            compiler_params=pltpu.CompilerParams(needs_layout_passes=False),
        )(_detile_sc_body))
    return _detile_sc_cache[0](*args)[0]


def _detile_tail(mT):
    return pl.pallas_call(
        _detile_body,
        grid=(TAIL_BLOCKS,),
        in_specs=[pl.BlockSpec((N_FACTORS, DT_C),
                               lambda i: (0, i + TAIL_BLOCK0))],
        out_specs=pl.BlockSpec((DT_C // 4, 128), lambda i: (i, 0)),
        out_shape=jax.ShapeDtypeStruct((ROWS_TAIL, 128), jnp.float32),
    )(mT)


def _gather_body(ulin, mlin, mtail, users_hbm, movies_hbm, uout, mout,
                 uidx_v, midx_v, ulane_v, mlane_v, mflag_v, rowidx_v, raw_v,
                 uex_v, mex_v, sem):
    wid = lax.axis_index("s") * NC + lax.axis_index("c")
    base = wid * B_PER_W
    pltpu.sync_copy(users_hbm.at[pl.ds(base, B_PER_W)], uidx_v)
    pltpu.sync_copy(movies_hbm.at[pl.ds(base, B_PER_W)], midx_v)

    iota = lax.iota(jnp.int32, 16)

    for g in range(B_PER_W // 16):
        sl = pl.ds(g * 16, 16)
        uvec = uidx_v[sl]
        rowidx_v[0, g // 8, pl.ds((g % 8) * 16, 16)] = \
            ((uvec >> 12) << 10) + (uvec & 1023)
        ulane_v[sl] = ((uvec >> 10) & 3) * 32
        mvec = midx_v[sl]
        tail = mvec >= SC_BOUND
        mrow = ((mvec >> 12) << 10) + (mvec & 1023)
        trow = (((mvec >> 12) - TAIL_BLOCK0) << 10) + (mvec & 1023)
        spread = g * 16 + iota
        rowidx_v[1, g // 8, pl.ds((g % 8) * 16, 16)] = \
            jnp.where(tail, spread, mrow)
        rowidx_v[2, g // 8, pl.ds((g % 8) * 16, 16)] = \
            jnp.where(tail, trow, spread)
        mlane_v[sl] = ((mvec >> 10) & 3) * 32
        mflag_v[sl] = jnp.where(tail, 1, 0)

    def gather_pass(table_sel, lin, lanes_v, ex_v, flag_val):
        copies = []
        for j in range(B_PER_W // 128):
            copies.append(pltpu.async_copy(
                lin.at[rowidx_v.at[table_sel, j]],
                raw_v.at[pl.ds(j * 128, 128)], sem))
        for c in copies:
            c.wait()

        def extract_f(f, carry):
            ng = B_PER_W // 16
            gathered = []
            for g in range(ng):
                rows16 = g * 16 + iota
                cols16 = lanes_v[pl.ds(g * 16, 16)] + f
                gathered.append(plsc.load_gather(raw_v, [rows16, cols16]))
            for g in range(ng):
                pos = (g * 16 + iota) * N_FACTORS + f
                if flag_val is None:
                    plsc.store_scatter(ex_v, [pos], gathered[g])
                else:
                    mask = mflag_v[pl.ds(g * 16, 16)] == flag_val
                    plsc.store_scatter(ex_v, [pos], gathered[g], mask=mask)
            return carry

        lax.fori_loop(0, N_FACTORS, extract_f, 0)

    gather_pass(0, ulin, ulane_v, uex_v, None)
    gather_pass(1, mlin, mlane_v, mex_v, 0)
    gather_pass(2, mtail, mlane_v, mex_v, 1)
    pltpu.sync_copy(uex_v, uout.at[wid])
    pltpu.sync_copy(mex_v, mout.at[wid])


_gather_cache = []


def _gather(*args):
    if not _gather_cache:
        _gather_cache.append(functools.partial(
            pl.kernel,
            mesh=plsc.VectorSubcoreMesh(core_axis_name="c",
                                        subcore_axis_name="s"),
            out_type=[
                jax.ShapeDtypeStruct((NW, PER_TEC), jnp.float32),
                jax.ShapeDtypeStruct((NW, PER_TEC), jnp.float32),
            ],
            scratch_types=[
                pltpu.VMEM((B_PER_W,), jnp.int32),
                pltpu.VMEM((B_PER_W,), jnp.int32),
                pltpu.VMEM((B_PER_W,), jnp.int32),
                pltpu.VMEM((B_PER_W,), jnp.int32),
                pltpu.VMEM((B_PER_W,), jnp.int32),
                pltpu.VMEM((3, B_PER_W // 128, 128), jnp.int32),
                pltpu.VMEM((B_PER_W, 128), jnp.float32),
                pltpu.VMEM((PER_TEC,), jnp.float32),
                pltpu.VMEM((PER_TEC,), jnp.float32),
                pltpu.SemaphoreType.DMA,
            ],
            compiler_params=pltpu.CompilerParams(needs_layout_passes=False),
        )(_gather_body))
    return _gather_cache[0](*args)


def _mlp_body(u_ref, m_ref, w1u_ref, w1m_ref, b1_ref, w2_ref, b2_ref,
              wf_ref, bf_ref, o_ref):
    x = jnp.dot(u_ref[...], w1u_ref[...], preferred_element_type=jnp.float32)
    x = x + jnp.dot(m_ref[...], w1m_ref[...], preferred_element_type=jnp.float32)
    h = jnp.maximum(x + b1_ref[...], 0.0)
    h = jnp.maximum(
        jnp.dot(h, w2_ref[...], preferred_element_type=jnp.float32)
        + b2_ref[...], 0.0)
    s = jnp.dot(h, wf_ref[...], preferred_element_type=jnp.float32) + bf_ref[...]
    o_ref[...] = jax.nn.sigmoid(s) * 4.5 + 0.5


def _mlp(u, m, w1u, w1m, b1, w2, b2, wf, bf, block_b=4096):
    nb = BATCH // block_b
    wspec = lambda shape: pl.BlockSpec(shape, lambda i: (0, 0))
    return pl.pallas_call(
        _mlp_body,
        grid=(nb,),
        in_specs=[
            pl.BlockSpec((block_b, N_FACTORS), lambda i: (i, 0)),
            pl.BlockSpec((block_b, N_FACTORS), lambda i: (i, 0)),
            wspec(w1u.shape),
            wspec(w1m.shape),
            wspec(b1.shape),
            wspec(w2.shape),
            wspec(b2.shape),
            wspec(wf.shape),
            wspec(bf.shape),
        ],
        out_specs=pl.BlockSpec((block_b, 1), lambda i: (i, 0)),
        out_shape=jax.ShapeDtypeStruct((BATCH, 1), jnp.float32),
    )(u, m, w1u, w1m, b1, w2, b2, wf, bf)


@jax.jit
def kernel(users, movies, user_emb, movie_emb, W1, b1, W2, b2, Wf, bf):
    u_lin = _detile(user_emb.T)
    m_lin = _detile_sc(movie_emb.T)
    m_tail = _detile_tail(movie_emb.T)
    u_raw, m_raw = _gather(u_lin, m_lin, m_tail,
                           users.astype(jnp.int32), movies.astype(jnp.int32))
    u_rows = u_raw.reshape(BATCH, N_FACTORS)
    m_rows = m_raw.reshape(BATCH, N_FACTORS)
    w1u = W1[:N_FACTORS]
    w1m = W1[N_FACTORS:]
    return _mlp(u_rows, m_rows, w1u, w1m,
                b1.reshape(1, -1), W2, b2.reshape(1, -1),
                Wf, bf.reshape(1, 1))


# speedup vs baseline: 2.5403x; 1.0003x over previous
import functools

import jax
import jax.numpy as jnp
from jax import lax
from jax.experimental import pallas as pl
from jax.experimental.pallas import tpu as pltpu
from jax.experimental.pallas import tpu_sc as plsc

N_FACTORS = 32
BATCH = 16384
N_ROWS = 1000000
NC = 2
NS = 16
NW = NC * NS
B_PER_W = BATCH // NW
PER_TEC = B_PER_W * N_FACTORS

DT_C = 4096
DT_GRID = (N_ROWS + DT_C - 1) // DT_C
ROWS_LIN = DT_GRID * (DT_C // 4)


def _detile_body(x_ref, o_ref):
    for q in range(4):
        xq = x_ref[:, 1024 * q: 1024 * (q + 1)]
        o_ref[:, 32 * q: 32 * (q + 1)] = xq.T


def _detile(xT):
    return pl.pallas_call(
        _detile_body,
        grid=(DT_GRID,),
        in_specs=[pl.BlockSpec((N_FACTORS, DT_C), lambda i: (0, i))],
        out_specs=pl.BlockSpec((DT_C // 4, 128), lambda i: (i, 0)),
        out_shape=jax.ShapeDtypeStruct((ROWS_LIN, 128), jnp.float32),
    )(xT)


SC_BOUND = 671744
SC_WINDOWS = SC_BOUND // 512
SC_W_PER_TEC = SC_WINDOWS // NW
ROWS_MAIN = (SC_BOUND // 4096) * 1024
TAIL_BLOCK0 = SC_BOUND // DT_C
TAIL_BLOCKS = (N_ROWS - SC_BOUND + DT_C - 1) // DT_C
ROWS_TAIL = TAIL_BLOCKS * (DT_C // 4)


def _detile_sc_body(mT, mlin, slab_v, st_v, sem_in, sem_out):
    wid = lax.axis_index("s") * NC + lax.axis_index("c")
    iota = lax.iota(jnp.int32, 16)

    def window_coords(t):
        wnd = wid * SC_W_PER_TEC + t
        k = wnd >> 3
        v0 = (wnd & 7) * 128
        return k, v0

    def fire_in(t, slot):
        k, v0 = window_coords(t)
        for q in range(4):
            c0 = pl.multiple_of(k * 4096 + q * 1024 + v0, 128)
            for tr in range(4):
                pltpu.async_copy(
                    mT.at[pl.ds(8 * tr, 8), pl.ds(c0, 128)],
                    slab_v.at[slot, q, pl.ds(8 * tr, 8), :], sem_in)

    def wait_in(t, slot):
        k, v0 = window_coords(t)
        for q in range(4):
            c0 = pl.multiple_of(k * 4096 + q * 1024 + v0, 128)
            for tr in range(4):
                pltpu.make_async_copy(
                    mT.at[pl.ds(8 * tr, 8), pl.ds(c0, 128)],
                    slab_v.at[slot, q, pl.ds(8 * tr, 8), :], sem_in).wait()

    def out_dst(t):
        k, v0 = window_coords(t)
        return mlin.at[pl.ds(pl.multiple_of(k * 1024 + v0, 128), 128), :]

    fire_in(0, 0)

    def body(t, carry):
        slot = t & 1

        @pl.when(t + 1 < SC_W_PER_TEC)
        def _():
            fire_in(t + 1, slot ^ 1)

        wait_in(t, slot)

        @pl.when(t >= 2)
        def _():
            pltpu.make_async_copy(st_v.at[slot], out_dst(t), sem_out).wait()

        for q in range(4):
            for f in range(N_FACTORS):
                col = jnp.full((16,), 32 * q + f, jnp.int32)
                vals = [slab_v[slot, q, f, pl.ds(g * 16, 16)]
                        for g in range(8)]
                for g in range(8):
                    plsc.store_scatter(st_v.at[slot],
                                       [g * 16 + iota, col], vals[g])
        pltpu.async_copy(st_v.at[slot], out_dst(t), sem_out)
        return carry

    lax.fori_loop(0, SC_W_PER_TEC, body, 0)
    for t in (SC_W_PER_TEC - 2, SC_W_PER_TEC - 1):
        pltpu.make_async_copy(st_v.at[t & 1], out_dst(t), sem_out).wait()


_detile_sc_cache = []


def _detile_sc(*args):
    if not _detile_sc_cache:
        _detile_sc_cache.append(functools.partial(
            pl.kernel,
            mesh=plsc.VectorSubcoreMesh(core_axis_name="c",
                                        subcore_axis_name="s"),
            out_type=[jax.ShapeDtypeStruct((ROWS_MAIN, 128), jnp.float32)],
            scratch_types=[
                pltpu.VMEM((2, 4, N_FACTORS, 128), jnp.float32),
                pltpu.VMEM((2, 128, 128), jnp.float32),
                pltpu.SemaphoreType.DMA,
                pltpu.SemaphoreType.DMA,
            ],
            compiler_params=pltpu.CompilerParams(needs_layout_passes=False),
        )(_detile_sc_body))
    return _detile_sc_cache[0](*args)[0]


def _detile_tail(mT):
    return pl.pallas_call(
        _detile_body,
        grid=(TAIL_BLOCKS,),
        in_specs=[pl.BlockSpec((N_FACTORS, DT_C),
                               lambda i: (0, i + TAIL_BLOCK0))],
        out_specs=pl.BlockSpec((DT_C // 4, 128), lambda i: (i, 0)),
        out_shape=jax.ShapeDtypeStruct((ROWS_TAIL, 128), jnp.float32),
    )(mT)


def _gather_body(ulin, mlin, mtail, users_hbm, movies_hbm, uout, mout,
                 uidx_v, midx_v, ulane_v, mlane_v, mflag_v, rowidx_v, raw_v,
                 uex_v, mex_v, sem):
    wid = lax.axis_index("s") * NC + lax.axis_index("c")
    base = wid * B_PER_W
    pltpu.sync_copy(users_hbm.at[pl.ds(base, B_PER_W)], uidx_v)
    pltpu.sync_copy(movies_hbm.at[pl.ds(base, B_PER_W)], midx_v)

    iota = lax.iota(jnp.int32, 16)

    for g in range(B_PER_W // 16):
        sl = pl.ds(g * 16, 16)
        uvec = uidx_v[sl]
        rowidx_v[0, g // 8, pl.ds((g % 8) * 16, 16)] = \
            ((uvec >> 12) << 10) + (uvec & 1023)
        ulane_v[sl] = ((uvec >> 10) & 3) * 32
        mvec = midx_v[sl]
        tail = mvec >= SC_BOUND
        mrow = ((mvec >> 12) << 10) + (mvec & 1023)
        trow = (((mvec >> 12) - TAIL_BLOCK0) << 10) + (mvec & 1023)
        spread = g * 16 + iota
        rowidx_v[1, g // 8, pl.ds((g % 8) * 16, 16)] = \
            jnp.where(tail, spread, mrow)
        rowidx_v[2, g // 8, pl.ds((g % 8) * 16, 16)] = \
            jnp.where(tail, trow, spread)
        mlane_v[sl] = ((mvec >> 10) & 3) * 32
        mflag_v[sl] = jnp.where(tail, 1, 0)

    def gather_pass(table_sel, lin, lanes_v, ex_v, flag_val):
        copies = []
        for j in range(B_PER_W // 128):
            copies.append(pltpu.async_copy(
                lin.at[rowidx_v.at[table_sel, j]],
                raw_v.at[pl.ds(j * 128, 128)], sem))
        for c in copies:
            c.wait()

        def extract_f(f, carry):
            ng = B_PER_W // 16
            gathered = []
            for g in range(ng):
                rows16 = g * 16 + iota
                cols16 = lanes_v[pl.ds(g * 16, 16)] + f
                gathered.append(plsc.load_gather(raw_v, [rows16, cols16]))
            for g in range(ng):
                pos = (g * 16 + iota) * N_FACTORS + f
                if flag_val is None:
                    plsc.store_scatter(ex_v, [pos], gathered[g])
                else:
                    mask = mflag_v[pl.ds(g * 16, 16)] == flag_val
                    plsc.store_scatter(ex_v, [pos], gathered[g], mask=mask)
            return carry

        lax.fori_loop(0, N_FACTORS, extract_f, 0)

    gather_pass(0, ulin, ulane_v, uex_v, None)
    gather_pass(1, mlin, mlane_v, mex_v, 0)
    gather_pass(2, mtail, mlane_v, mex_v, 1)
    pltpu.sync_copy(uex_v, uout.at[wid])
    pltpu.sync_copy(mex_v, mout.at[wid])


_gather_cache = []


def _gather(*args):
    if not _gather_cache:
        _gather_cache.append(functools.partial(
            pl.kernel,
            mesh=plsc.VectorSubcoreMesh(core_axis_name="c",
                                        subcore_axis_name="s"),
            out_type=[
                jax.ShapeDtypeStruct((NW, PER_TEC), jnp.float32),
                jax.ShapeDtypeStruct((NW, PER_TEC), jnp.float32),
            ],
            scratch_types=[
                pltpu.VMEM((B_PER_W,), jnp.int32),
                pltpu.VMEM((B_PER_W,), jnp.int32),
                pltpu.VMEM((B_PER_W,), jnp.int32),
                pltpu.VMEM((B_PER_W,), jnp.int32),
                pltpu.VMEM((B_PER_W,), jnp.int32),
                pltpu.VMEM((3, B_PER_W // 128, 128), jnp.int32),
                pltpu.VMEM((B_PER_W, 128), jnp.float32),
                pltpu.VMEM((PER_TEC,), jnp.float32),
                pltpu.VMEM((PER_TEC,), jnp.float32),
                pltpu.SemaphoreType.DMA,
            ],
            compiler_params=pltpu.CompilerParams(needs_layout_passes=False),
        )(_gather_body))
    return _gather_cache[0](*args)


def _mlp_body(u_ref, m_ref, w1u_ref, w1m_ref, b1_ref, w2_ref, b2_ref,
              wf_ref, bf_ref, o_ref):
    x = jnp.dot(u_ref[...], w1u_ref[...], preferred_element_type=jnp.float32)
    x = x + jnp.dot(m_ref[...], w1m_ref[...], preferred_element_type=jnp.float32)
    h = jnp.maximum(x + b1_ref[...], 0.0)
    h = jnp.maximum(
        jnp.dot(h, w2_ref[...], preferred_element_type=jnp.float32)
        + b2_ref[...], 0.0)
    s = jnp.dot(h, wf_ref[...], preferred_element_type=jnp.float32) + bf_ref[...]
    o_ref[...] = jax.nn.sigmoid(s) * 4.5 + 0.5


def _mlp(u, m, w1u, w1m, b1, w2, b2, wf, bf, block_b=4096):
    nb = BATCH // block_b
    wspec = lambda shape: pl.BlockSpec(shape, lambda i: (0, 0))
    return pl.pallas_call(
        _mlp_body,
        grid=(nb,),
        in_specs=[
            pl.BlockSpec((block_b, N_FACTORS), lambda i: (i, 0)),
            pl.BlockSpec((block_b, N_FACTORS), lambda i: (i, 0)),
            wspec(w1u.shape),
            wspec(w1m.shape),
            wspec(b1.shape),
            wspec(w2.shape),
            wspec(b2.shape),
            wspec(wf.shape),
            wspec(bf.shape),
        ],
        out_specs=pl.BlockSpec((block_b, 1), lambda i: (i, 0)),
        out_shape=jax.ShapeDtypeStruct((BATCH, 1), jnp.float32),
    )(u, m, w1u, w1m, b1, w2, b2, wf, bf)


@jax.jit
def kernel(users, movies, user_emb, movie_emb, W1, b1, W2, b2, Wf, bf):
    u_lin = _detile(user_emb.T)
    m_lin = _detile_sc(movie_emb.T)
    m_tail = _detile_tail(movie_emb.T)
    u_raw, m_raw = _gather(u_lin, m_lin, m_tail,
                           users.astype(jnp.int32), movies.astype(jnp.int32))
    u_rows = u_raw.reshape(BATCH, N_FACTORS)
    m_rows = m_raw.reshape(BATCH, N_FACTORS)
    w1u = W1[:N_FACTORS]
    w1m = W1[N_FACTORS:]
    return _mlp(u_rows, m_rows, w1u, w1m,
                b1.reshape(1, -1), W2, b2.reshape(1, -1),
                Wf, bf.reshape(1, 1))
